# baseline (speedup 1.0000x reference)
import functools

import jax
import jax.numpy as jnp
from jax import lax
from jax.experimental import pallas as pl
from jax.experimental.pallas import tpu as pltpu
from jax.experimental.pallas import tpu_sc as plsc

_NBINS = 10001
_NBPAD = 10240
_SLICE = _NBPAD // 16
_N = 100000
_NTILES = 16
_CHUNK = 6256
_LCHUNK = _N - _CHUNK * (_NTILES - 1)


def _auc_body(preds_hbm, targets_hbm, rows_hbm, out_hbm,
              preds_v, targets_v, hist_v, idx_v, tps_v, fps_v,
              stat_v, vec_v, out_v, shared, sums_sh, terms_sh):
    wid = lax.axis_index("s")
    base = wid * _CHUNK

    zeros = jnp.zeros((16,), jnp.float32)
    ones = jnp.ones((16,), jnp.float32)
    iota = lax.iota(jnp.int32, 16)
    zeros_i = jnp.zeros((16,), jnp.int32)

    def zinit(j, c):
        hist_v[0, pl.ds(j * 16, 16)] = zeros
        hist_v[1, pl.ds(j * 16, 16)] = zeros
        return c
    lax.fori_loop(0, _NBPAD // 16, zinit, 0, unroll=4)

    @pl.when(wid < _NTILES - 1)
    def _():
        pltpu.sync_copy(preds_hbm.at[pl.ds(base, _CHUNK)], preds_v)
        pltpu.sync_copy(targets_hbm.at[pl.ds(base, _CHUNK)], targets_v)

    @pl.when(wid == _NTILES - 1)
    def _():
        pltpu.sync_copy(preds_hbm.at[pl.ds(base, _LCHUNK)],
                        preds_v.at[pl.ds(0, _LCHUNK)])
        pltpu.sync_copy(targets_hbm.at[pl.ds(base, _LCHUNK)],
                        targets_v.at[pl.ds(0, _LCHUNK)])

    pltpu.sync_copy(rows_hbm, idx_v)

    @pl.when(wid == 0)
    def _():
        pltpu.sync_copy(hist_v, shared)

    def body(j, c):
        x = preds_v[pl.ds(j * 16, 16)]
        t = targets_v[pl.ds(j * 16, 16)]
        b = (10000.0 / (1.0 + jnp.exp(-x))).astype(jnp.int32)
        pos = t >= 0.5
        plsc.addupdate_scatter(hist_v.at[1], [b], ones, mask=pos)
        plsc.addupdate_scatter(hist_v.at[0], [b], ones,
                               mask=jnp.logical_not(pos))
        return c

    @pl.when(wid < _NTILES - 1)
    def _():
        lax.fori_loop(0, _CHUNK // 16, body, 0, unroll=4)

    @pl.when(wid == _NTILES - 1)
    def _():
        lax.fori_loop(0, _LCHUNK // 16, body, 0, unroll=4)

    plsc.subcore_barrier()
    pltpu.sync_copy(hist_v, shared.at[idx_v], add=True)
    plsc.subcore_barrier()

    sbase = wid * _SLICE
    pltpu.sync_copy(shared.at[1, pl.ds(sbase, _SLICE)], tps_v)
    pltpu.sync_copy(shared.at[0, pl.ds(sbase, _SLICE)], fps_v)

    def slsum(j, c):
        atp, afp = c
        return (atp + tps_v[pl.ds(j * 16, 16)], afp + fps_v[pl.ds(j * 16, 16)])
    atp, afp = lax.fori_loop(0, _SLICE // 16, slsum, (zeros, zeros), unroll=4)

    vec_v[...] = jnp.sum(atp) * ones
    pltpu.sync_copy(vec_v, sums_sh.at[1, wid])
    vec_v[...] = jnp.sum(afp) * ones
    pltpu.sync_copy(vec_v, sums_sh.at[0, wid])
    plsc.subcore_barrier()

    pltpu.sync_copy(sums_sh, stat_v)
    tp_sums = plsc.load_gather(stat_v, [jnp.full((16,), 1, jnp.int32),
                                        iota, zeros_i])
    fp_sums = plsc.load_gather(stat_v, [zeros_i, iota, zeros_i])
    s_tp = jnp.sum(tp_sums) * ones
    s_fp = jnp.sum(fp_sums) * ones
    rr = (ones / s_tp) * (ones / s_fp)
    my_off = jnp.sum(jnp.where(iota < wid, tp_sums, zeros))

    def trapz(j, c):
        cexcl, acc = c
        tpv = tps_v[pl.ds(j * 16, 16)]
        fpv = fps_v[pl.ds(j * 16, 16)]
        incl = plsc.cumsum(tpv)
        excl = cexcl + incl - tpv
        acc = acc + (s_tp - excl - 0.5 * tpv) * fpv
        return (cexcl + incl[15], acc)
    _, acc = lax.fori_loop(0, _SLICE // 16, trapz, (my_off, zeros))

    vec_v[...] = jnp.sum(acc * rr) * ones
    pltpu.sync_copy(vec_v, terms_sh.at[wid])
    plsc.subcore_barrier()

    @pl.when(wid == 0)
    def _():
        pltpu.sync_copy(terms_sh, stat_v.at[0])
        terms = plsc.load_gather(stat_v, [zeros_i, iota, zeros_i])
        out_v[...] = jnp.sum(terms) * ones
        pltpu.sync_copy(out_v, out_hbm)


@jax.jit
def _auc_call(preds, targets):
    mesh = plsc.VectorSubcoreMesh(core_axis_name="c", subcore_axis_name="s",
                                  num_cores=1)
    run = functools.partial(
        pl.kernel, mesh=mesh,
        compiler_params=pltpu.CompilerParams(use_tc_tiling_on_sc=False,
                                             needs_layout_passes=False),
        out_type=jax.ShapeDtypeStruct((16,), jnp.float32),
        scratch_types=[
            pltpu.VMEM((_CHUNK,), jnp.float32),
            pltpu.VMEM((_CHUNK,), jnp.float32),
            pltpu.VMEM((2, _NBPAD), jnp.float32),
            pltpu.VMEM((2,), jnp.int32),
            pltpu.VMEM((_SLICE,), jnp.float32),
            pltpu.VMEM((_SLICE,), jnp.float32),
            pltpu.VMEM((2, 16, 16), jnp.float32),
            pltpu.VMEM((16,), jnp.float32),
            pltpu.VMEM((16,), jnp.float32),
            pltpu.VMEM_SHARED((2, _NBPAD), jnp.float32),
            pltpu.VMEM_SHARED((2, 16, 16), jnp.float32),
            pltpu.VMEM_SHARED((16, 16), jnp.float32),
        ],
    )(_auc_body)
    return run(preds, targets, jnp.arange(2, dtype=jnp.int32))


def kernel(preds, targets):
    out = _auc_call(preds.reshape(-1), targets.reshape(-1))
    return out[0]


# speedup vs baseline: 1.3281x; 1.3281x over previous
import functools

import jax
import jax.numpy as jnp
from jax import lax
from jax.experimental import pallas as pl
from jax.experimental.pallas import tpu as pltpu
from jax.experimental.pallas import tpu_sc as plsc

_NBINS = 10001
_NBPAD = 10240
_SLICE = _NBPAD // 16
_N = 100000
_NTILES = 16
_CHUNK = 6256
_LCHUNK = _N - _CHUNK * (_NTILES - 1)


def _auc_body(preds_hbm, targets_hbm, rows_hbm, out_hbm,
              preds_v, targets_v, hist_v, idx_v, tps_v, fps_v,
              stat_v, vec_v, out_v, shared, sums_sh, terms_sh):
    wid = lax.axis_index("s")
    base = wid * _CHUNK

    zeros = jnp.zeros((16,), jnp.float32)
    ones = jnp.ones((16,), jnp.float32)
    iota = lax.iota(jnp.int32, 16)
    zeros_i = jnp.zeros((16,), jnp.int32)

    def zinit(j, c):
        hist_v[0, pl.ds(j * 16, 16)] = zeros
        hist_v[1, pl.ds(j * 16, 16)] = zeros
        return c
    lax.fori_loop(0, _NBPAD // 16, zinit, 0, unroll=4)

    @pl.when(wid < _NTILES - 1)
    def _():
        pltpu.sync_copy(preds_hbm.at[pl.ds(base, _CHUNK)], preds_v)
        pltpu.sync_copy(targets_hbm.at[pl.ds(base, _CHUNK)], targets_v)

    @pl.when(wid == _NTILES - 1)
    def _():
        pltpu.sync_copy(preds_hbm.at[pl.ds(base, _LCHUNK)],
                        preds_v.at[pl.ds(0, _LCHUNK)])
        pltpu.sync_copy(targets_hbm.at[pl.ds(base, _LCHUNK)],
                        targets_v.at[pl.ds(0, _LCHUNK)])

    pltpu.sync_copy(rows_hbm, idx_v)

    @pl.when(wid == 0)
    def _():
        pltpu.sync_copy(hist_v, shared)

    def body(j):
        x = preds_v[pl.ds(j * 16, 16)]
        t = targets_v[pl.ds(j * 16, 16)]
        b = (10000.0 / (1.0 + jnp.exp(-x))).astype(jnp.int32)
        pos = t >= 0.5
        plsc.addupdate_scatter(hist_v.at[1], [b], ones, mask=pos)
        plsc.addupdate_scatter(hist_v.at[0], [b], ones,
                               mask=jnp.logical_not(pos))

    @pl.when(wid < _NTILES - 1)
    def _():
        plsc.parallel_loop(0, _CHUNK // 16, unroll=4)(body)

    @pl.when(wid == _NTILES - 1)
    def _():
        plsc.parallel_loop(0, _LCHUNK // 16, unroll=4)(body)

    plsc.subcore_barrier()
    pltpu.sync_copy(hist_v, shared.at[idx_v], add=True)
    plsc.subcore_barrier()

    sbase = wid * _SLICE
    pltpu.sync_copy(shared.at[1, pl.ds(sbase, _SLICE)], tps_v)
    pltpu.sync_copy(shared.at[0, pl.ds(sbase, _SLICE)], fps_v)

    def slsum(j, c):
        atp, afp = c
        return (atp + tps_v[pl.ds(j * 16, 16)], afp + fps_v[pl.ds(j * 16, 16)])
    atp, afp = lax.fori_loop(0, _SLICE // 16, slsum, (zeros, zeros), unroll=4)

    vec_v[...] = jnp.sum(atp) * ones
    pltpu.sync_copy(vec_v, sums_sh.at[1, wid])
    vec_v[...] = jnp.sum(afp) * ones
    pltpu.sync_copy(vec_v, sums_sh.at[0, wid])
    plsc.subcore_barrier()

    pltpu.sync_copy(sums_sh, stat_v)
    tp_sums = plsc.load_gather(stat_v, [jnp.full((16,), 1, jnp.int32),
                                        iota, zeros_i])
    fp_sums = plsc.load_gather(stat_v, [zeros_i, iota, zeros_i])
    s_tp = jnp.sum(tp_sums) * ones
    s_fp = jnp.sum(fp_sums) * ones
    rr = (ones / s_tp) * (ones / s_fp)
    my_off = jnp.sum(jnp.where(iota < wid, tp_sums, zeros))

    def trapz(j, c):
        cexcl, acc = c
        tpv = tps_v[pl.ds(j * 16, 16)]
        fpv = fps_v[pl.ds(j * 16, 16)]
        incl = plsc.cumsum(tpv)
        excl = cexcl + incl - tpv
        acc = acc + (s_tp - excl - 0.5 * tpv) * fpv
        return (cexcl + incl[15], acc)
    _, acc = lax.fori_loop(0, _SLICE // 16, trapz, (my_off, zeros))

    vec_v[...] = jnp.sum(acc * rr) * ones
    pltpu.sync_copy(vec_v, terms_sh.at[wid])
    plsc.subcore_barrier()

    @pl.when(wid == 0)
    def _():
        pltpu.sync_copy(terms_sh, stat_v.at[0])
        terms = plsc.load_gather(stat_v, [zeros_i, iota, zeros_i])
        out_v[...] = jnp.sum(terms) * ones
        pltpu.sync_copy(out_v, out_hbm)


@jax.jit
def _auc_call(preds, targets):
    mesh = plsc.VectorSubcoreMesh(core_axis_name="c", subcore_axis_name="s",
                                  num_cores=1)
    run = functools.partial(
        pl.kernel, mesh=mesh,
        compiler_params=pltpu.CompilerParams(use_tc_tiling_on_sc=False,
                                             needs_layout_passes=False),
        out_type=jax.ShapeDtypeStruct((16,), jnp.float32),
        scratch_types=[
            pltpu.VMEM((_CHUNK,), jnp.float32),
            pltpu.VMEM((_CHUNK,), jnp.float32),
            pltpu.VMEM((2, _NBPAD), jnp.float32),
            pltpu.VMEM((2,), jnp.int32),
            pltpu.VMEM((_SLICE,), jnp.float32),
            pltpu.VMEM((_SLICE,), jnp.float32),
            pltpu.VMEM((2, 16, 16), jnp.float32),
            pltpu.VMEM((16,), jnp.float32),
            pltpu.VMEM((16,), jnp.float32),
            pltpu.VMEM_SHARED((2, _NBPAD), jnp.float32),
            pltpu.VMEM_SHARED((2, 16, 16), jnp.float32),
            pltpu.VMEM_SHARED((16, 16), jnp.float32),
        ],
    )(_auc_body)
    return run(preds, targets, jnp.arange(2, dtype=jnp.int32))


def kernel(preds, targets):
    out = _auc_call(preds.reshape(-1), targets.reshape(-1))
    return out[0]


# speedup vs baseline: 1.3416x; 1.0101x over previous
import functools

import jax
import jax.numpy as jnp
from jax import lax
from jax.experimental import pallas as pl
from jax.experimental.pallas import tpu as pltpu
from jax.experimental.pallas import tpu_sc as plsc

_NBINS = 10001
_NBPAD = 10240
_SLICE = _NBPAD // 16
_N = 100000
_NTILES = 16
_CHUNK = 6256
_LCHUNK = _N - _CHUNK * (_NTILES - 1)


def _auc_body(preds_hbm, targets_hbm, rows_hbm, out_hbm,
              preds_v, targets_v, hist_v, idx_v, tps_v, fps_v,
              stat_v, vec_v, out_v, shared, sums_sh, terms_sh):
    wid = lax.axis_index("s")
    base = wid * _CHUNK

    zeros = jnp.zeros((16,), jnp.float32)
    ones = jnp.ones((16,), jnp.float32)
    iota = lax.iota(jnp.int32, 16)
    zeros_i = jnp.zeros((16,), jnp.int32)

    @plsc.parallel_loop(0, _NBPAD // 16, unroll=8)
    def _(j):
        hist_v[0, pl.ds(j * 16, 16)] = zeros
        hist_v[1, pl.ds(j * 16, 16)] = zeros

    @pl.when(wid < _NTILES - 1)
    def _():
        pltpu.sync_copy(preds_hbm.at[pl.ds(base, _CHUNK)], preds_v)
        pltpu.sync_copy(targets_hbm.at[pl.ds(base, _CHUNK)], targets_v)

    @pl.when(wid == _NTILES - 1)
    def _():
        pltpu.sync_copy(preds_hbm.at[pl.ds(base, _LCHUNK)],
                        preds_v.at[pl.ds(0, _LCHUNK)])
        pltpu.sync_copy(targets_hbm.at[pl.ds(base, _LCHUNK)],
                        targets_v.at[pl.ds(0, _LCHUNK)])

    pltpu.sync_copy(rows_hbm, idx_v)

    @pl.when(wid == 0)
    def _():
        pltpu.sync_copy(hist_v, shared)

    def body(j):
        x = preds_v[pl.ds(j * 16, 16)]
        t = targets_v[pl.ds(j * 16, 16)]
        b = (10000.0 / (1.0 + jnp.exp(-x))).astype(jnp.int32)
        pos = t >= 0.5
        plsc.addupdate_scatter(hist_v.at[1], [b], ones, mask=pos)
        plsc.addupdate_scatter(hist_v.at[0], [b], ones,
                               mask=jnp.logical_not(pos))

    @pl.when(wid < _NTILES - 1)
    def _():
        plsc.parallel_loop(0, _CHUNK // 16, unroll=4)(body)

    @pl.when(wid == _NTILES - 1)
    def _():
        plsc.parallel_loop(0, _LCHUNK // 16, unroll=4)(body)

    plsc.subcore_barrier()
    pltpu.sync_copy(hist_v, shared.at[idx_v], add=True)
    plsc.subcore_barrier()

    sbase = wid * _SLICE
    pltpu.sync_copy(shared.at[1, pl.ds(sbase, _SLICE)], tps_v)
    pltpu.sync_copy(shared.at[0, pl.ds(sbase, _SLICE)], fps_v)

    @plsc.parallel_loop(0, _SLICE // 16, unroll=4, carry=(zeros, zeros))
    def _slsum(j, c):
        atp, afp = c
        return (atp + tps_v[pl.ds(j * 16, 16)], afp + fps_v[pl.ds(j * 16, 16)])
    atp, afp = _slsum

    vec_v[...] = jnp.sum(atp) * ones
    pltpu.sync_copy(vec_v, sums_sh.at[1, wid])
    vec_v[...] = jnp.sum(afp) * ones
    pltpu.sync_copy(vec_v, sums_sh.at[0, wid])
    plsc.subcore_barrier()

    pltpu.sync_copy(sums_sh, stat_v)
    tp_sums = plsc.load_gather(stat_v, [jnp.full((16,), 1, jnp.int32),
                                        iota, zeros_i])
    fp_sums = plsc.load_gather(stat_v, [zeros_i, iota, zeros_i])
    s_tp = jnp.sum(tp_sums) * ones
    s_fp = jnp.sum(fp_sums) * ones
    rr = (ones / s_tp) * (ones / s_fp)
    my_off = jnp.sum(jnp.where(iota < wid, tp_sums, zeros))

    @plsc.parallel_loop(0, _SLICE // 16, unroll=4, carry=(my_off, zeros))
    def _trapz(j, c):
        cexcl, acc = c
        tpv = tps_v[pl.ds(j * 16, 16)]
        fpv = fps_v[pl.ds(j * 16, 16)]
        incl = plsc.cumsum(tpv)
        excl = cexcl + incl - tpv
        acc = acc + (s_tp - excl - 0.5 * tpv) * fpv
        return (cexcl + incl[15], acc)
    _, acc = _trapz

    vec_v[...] = jnp.sum(acc * rr) * ones
    pltpu.sync_copy(vec_v, terms_sh.at[wid])
    plsc.subcore_barrier()

    @pl.when(wid == 0)
    def _():
        pltpu.sync_copy(terms_sh, stat_v.at[0])
        terms = plsc.load_gather(stat_v, [zeros_i, iota, zeros_i])
        out_v[...] = jnp.sum(terms) * ones
        pltpu.sync_copy(out_v, out_hbm)


@jax.jit
def _auc_call(preds, targets):
    mesh = plsc.VectorSubcoreMesh(core_axis_name="c", subcore_axis_name="s",
                                  num_cores=1)
    run = functools.partial(
        pl.kernel, mesh=mesh,
        compiler_params=pltpu.CompilerParams(use_tc_tiling_on_sc=False,
                                             needs_layout_passes=False),
        out_type=jax.ShapeDtypeStruct((16,), jnp.float32),
        scratch_types=[
            pltpu.VMEM((_CHUNK,), jnp.float32),
            pltpu.VMEM((_CHUNK,), jnp.float32),
            pltpu.VMEM((2, _NBPAD), jnp.float32),
            pltpu.VMEM((2,), jnp.int32),
            pltpu.VMEM((_SLICE,), jnp.float32),
            pltpu.VMEM((_SLICE,), jnp.float32),
            pltpu.VMEM((2, 16, 16), jnp.float32),
            pltpu.VMEM((16,), jnp.float32),
            pltpu.VMEM((16,), jnp.float32),
            pltpu.VMEM_SHARED((2, _NBPAD), jnp.float32),
            pltpu.VMEM_SHARED((2, 16, 16), jnp.float32),
            pltpu.VMEM_SHARED((16, 16), jnp.float32),
        ],
    )(_auc_body)
    return run(preds, targets, jnp.arange(2, dtype=jnp.int32))


def kernel(preds, targets):
    out = _auc_call(preds.reshape(-1), targets.reshape(-1))
    return out[0]


# speedup vs baseline: 1.3644x; 1.0171x over previous
import functools

import jax
import jax.numpy as jnp
from jax import lax
from jax.experimental import pallas as pl
from jax.experimental.pallas import tpu as pltpu
from jax.experimental.pallas import tpu_sc as plsc

_NBINS = 10001
_NBPAD = 10240
_SLICE = _NBPAD // 16
_N = 100000
_NTILES = 16
_CHUNK = 6256
_LCHUNK = _N - _CHUNK * (_NTILES - 1)


def _auc_body(preds_hbm, targets_hbm, rows_hbm, out_hbm,
              preds_v, targets_v, hist_v, idx_v, tps_v, fps_v,
              stat_v, vec_v, out_v, shared, terms_sh):
    wid = lax.axis_index("s")
    base = wid * _CHUNK

    zeros = jnp.zeros((16,), jnp.float32)
    ones = jnp.ones((16,), jnp.float32)
    iota = lax.iota(jnp.int32, 16)
    zeros_i = jnp.zeros((16,), jnp.int32)

    @plsc.parallel_loop(0, _NBPAD // 16, unroll=8)
    def _(j):
        hist_v[0, pl.ds(j * 16, 16)] = zeros
        hist_v[1, pl.ds(j * 16, 16)] = zeros

    @pl.when(wid < _NTILES - 1)
    def _():
        pltpu.sync_copy(preds_hbm.at[pl.ds(base, _CHUNK)], preds_v)
        pltpu.sync_copy(targets_hbm.at[pl.ds(base, _CHUNK)], targets_v)

    @pl.when(wid == _NTILES - 1)
    def _():
        pltpu.sync_copy(preds_hbm.at[pl.ds(base, _LCHUNK)],
                        preds_v.at[pl.ds(0, _LCHUNK)])
        pltpu.sync_copy(targets_hbm.at[pl.ds(base, _LCHUNK)],
                        targets_v.at[pl.ds(0, _LCHUNK)])

    pltpu.sync_copy(rows_hbm, idx_v)

    @pl.when(wid == 0)
    def _():
        pltpu.sync_copy(hist_v, shared)

    def body(j):
        x = preds_v[pl.ds(j * 16, 16)]
        t = targets_v[pl.ds(j * 16, 16)]
        b = (10000.0 / (1.0 + jnp.exp(-x))).astype(jnp.int32)
        b2 = b + jnp.where(t >= 0.5, _NBPAD, 0).astype(jnp.int32)
        plsc.addupdate_scatter(hist_v.at[0], [b2], ones)

    @pl.when(wid < _NTILES - 1)
    def _():
        plsc.parallel_loop(0, _CHUNK // 16, unroll=4)(body)

    @pl.when(wid == _NTILES - 1)
    def _():
        plsc.parallel_loop(0, _LCHUNK // 16, unroll=4)(body)

    plsc.subcore_barrier()
    pltpu.sync_copy(hist_v, shared.at[idx_v], add=True)
    plsc.subcore_barrier()

    sbase = wid * _SLICE
    pltpu.sync_copy(shared.at[1, pl.ds(sbase, _SLICE)], tps_v)
    pltpu.sync_copy(shared.at[0, pl.ds(sbase, _SLICE)], fps_v)

    @plsc.parallel_loop(0, _SLICE // 16, unroll=4,
                        carry=(jnp.float32(0.0), zeros, zeros))
    def _scan(j, c):
        cloc, afp, accp = c
        tpv = tps_v[pl.ds(j * 16, 16)]
        fpv = fps_v[pl.ds(j * 16, 16)]
        incl = plsc.cumsum(tpv)
        lexcl = cloc + incl - tpv
        return (cloc + incl[15], afp + fpv, accp + (lexcl + 0.5 * tpv) * fpv)
    atp_s, afp_v, accp_v = _scan

    afp_s = jnp.sum(afp_v)
    par_s = jnp.sum(accp_v)
    vec_v[...] = jnp.where(iota == 0, atp_s,
                           jnp.where(iota == 1, afp_s, par_s))
    pltpu.sync_copy(vec_v, terms_sh.at[wid])
    plsc.subcore_barrier()

    @pl.when(wid == 0)
    def _():
        pltpu.sync_copy(terms_sh, stat_v)
        ones_i = jnp.full((16,), 1, jnp.int32)
        atp_t = plsc.load_gather(stat_v, [iota, zeros_i])
        afp_t = plsc.load_gather(stat_v, [iota, ones_i])
        par_t = plsc.load_gather(stat_v, [iota, ones_i + ones_i])
        s_tp = jnp.sum(atp_t) * ones
        s_fp = jnp.sum(afp_t) * ones
        off = plsc.cumsum(atp_t) - atp_t
        v = (s_tp - off) * afp_t - par_t
        out_v[...] = (jnp.sum(v) * ones) / (s_tp * s_fp)
        pltpu.sync_copy(out_v, out_hbm)


@jax.jit
def _auc_call(preds, targets):
    mesh = plsc.VectorSubcoreMesh(core_axis_name="c", subcore_axis_name="s",
                                  num_cores=1)
    run = functools.partial(
        pl.kernel, mesh=mesh,
        compiler_params=pltpu.CompilerParams(use_tc_tiling_on_sc=False,
                                             needs_layout_passes=False),
        out_type=jax.ShapeDtypeStruct((16,), jnp.float32),
        scratch_types=[
            pltpu.VMEM((_CHUNK,), jnp.float32),
            pltpu.VMEM((_CHUNK,), jnp.float32),
            pltpu.VMEM((2, _NBPAD), jnp.float32),
            pltpu.VMEM((2,), jnp.int32),
            pltpu.VMEM((_SLICE,), jnp.float32),
            pltpu.VMEM((_SLICE,), jnp.float32),
            pltpu.VMEM((16, 16), jnp.float32),
            pltpu.VMEM((16,), jnp.float32),
            pltpu.VMEM((16,), jnp.float32),
            pltpu.VMEM_SHARED((2, _NBPAD), jnp.float32),
            pltpu.VMEM_SHARED((16, 16), jnp.float32),
        ],
    )(_auc_body)
    return run(preds, targets, jnp.arange(2, dtype=jnp.int32))


def kernel(preds, targets):
    out = _auc_call(preds.reshape(-1), targets.reshape(-1))
    return out[0]
